# Initial kernel scaffold; baseline (speedup 1.0000x reference)
import jax
import jax.numpy as jnp
from jax.experimental import pallas as pl


def kernel(memory, n_id):
    raise NotImplementedError("write your pallas kernel here")


# speedup vs baseline: 1.5371x; 1.5371x over previous
import functools

import jax
import jax.numpy as jnp
from jax import lax
from jax.experimental import pallas as pl
from jax.experimental.pallas import tpu as pltpu
from jax.experimental.pallas import tpu_sc as plsc

D = 128
B = 16384
NC = 2
NS = 16
NW = NC * NS
BPW = B // NW
CH = 128
NCH = BPW // CH


def _gather_body(mem_hbm, idx_hbm, out_hbm, idx_v, rows_v, sem):
    wid = lax.axis_index("s") * NC + lax.axis_index("c")
    base = wid * BPW
    pltpu.sync_copy(idx_hbm.at[wid], idx_v)
    copies = [
        pltpu.async_copy(
            mem_hbm.at[idx_v.at[j]], rows_v.at[pl.ds(j * CH, CH)], sem
        )
        for j in range(NCH)
    ]
    for j in range(NCH):
        copies[j].wait()
        pltpu.sync_copy(
            rows_v.at[pl.ds(j * CH, CH)], out_hbm.at[pl.ds(base + j * CH, CH)]
        )


@jax.jit
def kernel(memory, n_id):
    mesh = plsc.VectorSubcoreMesh(
        core_axis_name="c", subcore_axis_name="s", num_cores=NC, num_subcores=NS
    )
    gather = functools.partial(
        pl.kernel,
        out_type=jax.ShapeDtypeStruct((B, D), jnp.float32),
        mesh=mesh,
        scratch_types=[
            pltpu.VMEM((NCH, CH), jnp.int32),
            pltpu.VMEM((BPW, D), jnp.float32),
            pltpu.SemaphoreType.DMA,
        ],
    )(_gather_body)
    idx = n_id.astype(jnp.int32).reshape(NW, NCH, CH)
    return gather(memory, idx)


# speedup vs baseline: 1.5421x; 1.0033x over previous
import functools

import jax
import jax.numpy as jnp
from jax import lax
from jax.experimental import pallas as pl
from jax.experimental.pallas import tpu as pltpu
from jax.experimental.pallas import tpu_sc as plsc

D = 128
B = 16384
NC = 2
NS = 16
NW = NC * NS
BPW = B // NW
CH = 128
NCH = BPW // CH


def _gather_body(mem_hbm, idx_hbm, out_hbm, idx_v, rows_v, gsem, wsem):
    wid = lax.axis_index("s") * NC + lax.axis_index("c")
    base = wid * BPW
    pltpu.sync_copy(idx_hbm.at[wid], idx_v)
    copies = [
        pltpu.async_copy(
            mem_hbm.at[idx_v.at[j]], rows_v.at[pl.ds(j * CH, CH)], gsem
        )
        for j in range(NCH)
    ]
    writes = []
    for j in range(NCH):
        copies[j].wait()
        writes.append(
            pltpu.async_copy(
                rows_v.at[pl.ds(j * CH, CH)],
                out_hbm.at[pl.ds(base + j * CH, CH)],
                wsem,
            )
        )
    for w in writes:
        w.wait()


@jax.jit
def kernel(memory, n_id):
    mesh = plsc.VectorSubcoreMesh(
        core_axis_name="c", subcore_axis_name="s", num_cores=NC, num_subcores=NS
    )
    gather = functools.partial(
        pl.kernel,
        out_type=jax.ShapeDtypeStruct((B, D), jnp.float32),
        mesh=mesh,
        scratch_types=[
            pltpu.VMEM((NCH, CH), jnp.int32),
            pltpu.VMEM((BPW, D), jnp.float32),
            pltpu.SemaphoreType.DMA,
            pltpu.SemaphoreType.DMA,
        ],
    )(_gather_body)
    idx = n_id.astype(jnp.int32).reshape(NW, NCH, CH)
    return gather(memory, idx)
